# baseline (speedup 1.0000x reference)
import jax
import jax.numpy as jnp
from jax import lax
from jax.experimental import pallas as pl
from jax.experimental.pallas import tpu as pltpu
from jax.experimental.pallas import tpu_sc as plsc

HIDDEN = 64
BATCH = 16384
NUM_CORES = 2
NUM_SUBCORES = 16
NW = NUM_CORES * NUM_SUBCORES
B_PER_W = BATCH // NW
CHUNK = 128
N_CHUNKS = B_PER_W // CHUNK
GROUPS = B_PER_W // 16


def _sc_body(labels_hbm, table_hbm, out_hbm, idx_v, rows_v, sem):
    wid = lax.axis_index("s") * NUM_CORES + lax.axis_index("c")
    base = wid * B_PER_W

    pltpu.sync_copy(labels_hbm.at[wid], idx_v)

    copies = [
        pltpu.async_copy(
            table_hbm.at[idx_v.at[ch]],
            rows_v.at[pl.ds(ch * CHUNK, CHUNK)],
            sem,
        )
        for ch in range(N_CHUNKS)
    ]
    for cp in copies:
        cp.wait()

    row_iota = lax.iota(jnp.int32, 16)

    def group(g, carry):
        rowv = g * 16 + row_iota
        acc = jnp.zeros((16,), jnp.float32)
        for j in range(HIDDEN):
            colv = jnp.full((16,), j, jnp.int32)
            v = plsc.load_gather(rows_v, [rowv, colv])
            acc = acc + v * v
        xi = plsc.bitcast(acc, jnp.int32)
        y = plsc.bitcast(jnp.int32(0x5F3759DF) - (xi >> 1), jnp.float32)
        for _ in range(3):
            y = y * (1.5 - 0.5 * acc * y * y)
        scale = jnp.minimum(y, 1.0)
        for j in range(HIDDEN):
            colv = jnp.full((16,), j, jnp.int32)
            v = plsc.load_gather(rows_v, [rowv, colv])
            plsc.store_scatter(rows_v, [rowv, colv], v * scale)
        return carry

    lax.fori_loop(0, GROUPS, group, 0)

    pltpu.sync_copy(rows_v, out_hbm.at[pl.ds(base, B_PER_W)])


def kernel(labels, embedding_table):
    labels_i = labels.astype(jnp.int32).reshape(NW, N_CHUNKS, CHUNK)
    run = pl.kernel(
        _sc_body,
        out_type=jax.ShapeDtypeStruct((BATCH, HIDDEN), jnp.float32),
        mesh=plsc.VectorSubcoreMesh(core_axis_name="c", subcore_axis_name="s"),
        compiler_params=pltpu.CompilerParams(
            needs_layout_passes=False, use_tc_tiling_on_sc=False
        ),
        scratch_types=[
            pltpu.VMEM((N_CHUNKS, CHUNK), jnp.int32),
            pltpu.VMEM((B_PER_W, HIDDEN), jnp.float32),
            pltpu.SemaphoreType.DMA,
        ],
    )
    return run(labels_i, embedding_table)


# speedup vs baseline: 1.0323x; 1.0323x over previous
import jax
import jax.numpy as jnp
from jax import lax
from jax.experimental import pallas as pl
from jax.experimental.pallas import tpu as pltpu
from jax.experimental.pallas import tpu_sc as plsc

HIDDEN = 64
BATCH = 16384
PAIR_ROWS = 500000
NUM_CORES = 2
NUM_SUBCORES = 16
NW = NUM_CORES * NUM_SUBCORES
B_PER_W = BATCH // NW
CHUNKS = B_PER_W // 16
GCHUNK = 128


def _sc_body(labels_hbm, pairs_hbm, out_hbm, lab_v, idx_v, gat_v, out_v, sem):
    wid = lax.axis_index("s") * NUM_CORES + lax.axis_index("c")
    base = wid * B_PER_W

    pltpu.sync_copy(labels_hbm, lab_v)

    lanes = lax.iota(jnp.int32, 16)

    def addr_chunk(ci, carry):
        lab = lab_v[pl.ds(base + ci * 16, 16)]
        idx_v[pl.ds(ci * 16, 16)] = lab >> 1
        return carry

    lax.fori_loop(0, CHUNKS, addr_chunk, 0)

    copies = [
        pltpu.async_copy(
            pairs_hbm.at[idx_v.at[pl.ds(c * GCHUNK, GCHUNK)]],
            gat_v.at[pl.ds(c * GCHUNK, GCHUNK), :],
            sem,
        )
        for c in range(B_PER_W // GCHUNK)
    ]
    for cp in copies:
        cp.wait()

    def chunk(ci, carry):
        lab = lab_v[pl.ds(base + ci * 16, 16)]
        half = (lab & 1) << 6
        rows = ci * 16 + lanes
        acc = jnp.zeros((16,), jnp.float32)
        for r in range(HIDDEN):
            v = plsc.load_gather(gat_v, [rows, half + r])
            acc = acc + v * v
        xi = plsc.bitcast(acc, jnp.int32)
        y = plsc.bitcast(jnp.int32(0x5F3759DF) - (xi >> 1), jnp.float32)
        for _ in range(3):
            y = y * (1.5 - 0.5 * acc * y * y)
        scale = jnp.minimum(y, 1.0)
        for r in range(HIDDEN):
            v = plsc.load_gather(gat_v, [rows, half + r])
            out_v[r, pl.ds(ci * 16, 16)] = v * scale
        return carry

    lax.fori_loop(0, CHUNKS, chunk, 0)

    pltpu.sync_copy(out_v, out_hbm.at[:, pl.ds(base, B_PER_W)])


def kernel(labels, embedding_table):
    labels_i = labels.astype(jnp.int32)
    pairs = embedding_table.reshape(PAIR_ROWS, 2 * HIDDEN)
    run = pl.kernel(
        _sc_body,
        out_type=jax.ShapeDtypeStruct((HIDDEN, BATCH), jnp.float32),
        mesh=plsc.VectorSubcoreMesh(core_axis_name="c", subcore_axis_name="s"),
        scratch_types=[
            pltpu.VMEM((BATCH,), jnp.int32),
            pltpu.VMEM((B_PER_W,), jnp.int32),
            pltpu.VMEM((B_PER_W, 2 * HIDDEN), jnp.float32),
            pltpu.VMEM((HIDDEN, B_PER_W), jnp.float32),
            pltpu.SemaphoreType.DMA,
        ],
        compiler_params=pltpu.CompilerParams(
            use_tc_tiling_on_sc=True, needs_layout_passes=False
        ),
    )
    out_t = run(labels_i, pairs)
    return out_t.T


# speedup vs baseline: 3.2272x; 3.1262x over previous
import jax
import jax.numpy as jnp
from jax import lax
from jax.experimental import pallas as pl
from jax.experimental.pallas import tpu as pltpu
from jax.experimental.pallas import tpu_sc as plsc

HIDDEN = 64
BATCH = 16384
NUM_CLASSES = 1000000
NUM_CORES = 2
NUM_SUBCORES = 16
NW = NUM_CORES * NUM_SUBCORES
B_PER_W = BATCH // NW
CHUNKS = B_PER_W // 16
NBUF = 8


def _sc_body(labels_hbm, tablet_hbm, out_hbm, lab_v, buf_v, out_v, sems):
    wid = lax.axis_index("s") * NUM_CORES + lax.axis_index("c")
    blk = pl.multiple_of((wid // 2) * 1024, 1024)
    pltpu.sync_copy(labels_hbm.at[pl.ds(blk, 1024)], lab_v)
    off = (wid % 2) * 512

    lanes = lax.iota(jnp.int32, 16)

    def fire(l, sl):
        start = pl.multiple_of((l >> 7) * 128, 128)
        pltpu.async_copy(
            tablet_hbm.at[:, pl.ds(start, 128)], buf_v.at[sl], sems.at[sl]
        )

    lv0 = lab_v[pl.ds(off, 16)]
    for j in range(NBUF):
        fire(lv0[j], j)

    def process(l, sl, iv):
        pltpu.make_async_copy(
            tablet_hbm.at[:, pl.ds(0, 128)], buf_v.at[sl], sems.at[sl]
        ).wait()
        colv = jnp.full((16,), l & 127, jnp.int32)
        sv = jnp.full((16,), sl, jnp.int32)
        vs = [
            plsc.load_gather(buf_v, [sv, lanes + 16 * k, colv])
            for k in range(4)
        ]
        n2 = jnp.sum(vs[0] * vs[0] + vs[1] * vs[1] + vs[2] * vs[2]
                     + vs[3] * vs[3])
        acc = jnp.full((16,), n2, jnp.float32)
        xi = plsc.bitcast(acc, jnp.int32)
        y = plsc.bitcast(jnp.int32(0x5F3759DF) - (xi >> 1), jnp.float32)
        for _ in range(3):
            y = y * (1.5 - 0.5 * acc * y * y)
        scale = jnp.minimum(y, 1.0)
        for k in range(4):
            plsc.store_scatter(out_v, [lanes + 16 * k, iv], vs[k] * scale)

    def chunk(ci, carry):
        lv = lab_v[pl.ds(off + ci * 16, 16)]
        for j in range(16):
            sl = j % NBUF
            iv = jnp.full((16,), ci * 16 + j, jnp.int32)
            process(lv[j], sl, iv)
            if j < NBUF:
                fire(lv[j + NBUF], sl)
            else:
                @pl.when(ci < CHUNKS - 1)
                def _():
                    lvn = lab_v[pl.ds(off + ci * 16 + 16, 16)]
                    fire(lvn[j - NBUF], sl)
        return carry

    lax.fori_loop(0, CHUNKS, chunk, 0)

    base = wid * B_PER_W
    pltpu.sync_copy(out_v, out_hbm.at[:, pl.ds(base, B_PER_W)])


def kernel(labels, embedding_table):
    labels_i = labels.astype(jnp.int32)
    run = pl.kernel(
        _sc_body,
        out_type=jax.ShapeDtypeStruct((HIDDEN, BATCH), jnp.float32),
        mesh=plsc.VectorSubcoreMesh(core_axis_name="c", subcore_axis_name="s"),
        scratch_types=[
            pltpu.VMEM((1024,), jnp.int32),
            pltpu.VMEM((NBUF, HIDDEN, 128), jnp.float32),
            pltpu.VMEM((HIDDEN, B_PER_W), jnp.float32),
            pltpu.SemaphoreType.DMA((NBUF,)),
        ],
        compiler_params=pltpu.CompilerParams(
            use_tc_tiling_on_sc=True, needs_layout_passes=False
        ),
    )
    out_t = run(labels_i, embedding_table.T)
    return out_t.T


# speedup vs baseline: 3.2668x; 1.0123x over previous
import jax
import jax.numpy as jnp
from jax import lax
from jax.experimental import pallas as pl
from jax.experimental.pallas import tpu as pltpu
from jax.experimental.pallas import tpu_sc as plsc

HIDDEN = 64
BATCH = 16384
NUM_CLASSES = 1000000
NUM_CORES = 2
NUM_SUBCORES = 16
NW = NUM_CORES * NUM_SUBCORES
B_PER_W = BATCH // NW
CHUNKS = B_PER_W // 16
NBUF = 8


def _sc_body(labels_hbm, tablet_hbm, out_hbm, lab_v, buf_v, out_v, sems):
    wid = lax.axis_index("s") * NUM_CORES + lax.axis_index("c")
    blk = pl.multiple_of((wid // 2) * 1024, 1024)
    pltpu.sync_copy(labels_hbm.at[pl.ds(blk, 1024)], lab_v)
    off = (wid % 2) * 512

    lanes = lax.iota(jnp.int32, 16)

    def fire(l, sl):
        start = pl.multiple_of((l >> 7) * 128, 128)
        pltpu.async_copy(
            tablet_hbm.at[:, pl.ds(start, 128)], buf_v.at[sl], sems.at[sl]
        )

    lv0 = lab_v[pl.ds(off, 16)]
    for j in range(NBUF):
        fire(lv0[j], j)

    def load_cols(l, sl):
        pltpu.make_async_copy(
            tablet_hbm.at[:, pl.ds(0, 128)], buf_v.at[sl], sems.at[sl]
        ).wait()
        colv = jnp.full((16,), l & 127, jnp.int32)
        sv = jnp.full((16,), sl, jnp.int32)
        return [
            plsc.load_gather(buf_v, [sv, lanes + 16 * k, colv])
            for k in range(4)
        ]

    def scale_store(vs, iv):
        n2 = jnp.sum(vs[0] * vs[0] + vs[1] * vs[1] + vs[2] * vs[2]
                     + vs[3] * vs[3])
        acc = jnp.full((16,), n2, jnp.float32)
        xi = plsc.bitcast(acc, jnp.int32)
        y = plsc.bitcast(jnp.int32(0x5F3759DF) - (xi >> 1), jnp.float32)
        for _ in range(3):
            y = y * (1.5 - 0.5 * acc * y * y)
        scale = jnp.minimum(y, 1.0)
        for k in range(4):
            plsc.store_scatter(out_v, [lanes + 16 * k, iv], vs[k] * scale)

    def chunk(ci, carry):
        lv = lab_v[pl.ds(off + ci * 16, 16)]
        for j in range(16):
            sl = j % NBUF
            iv = jnp.full((16,), ci * 16 + j, jnp.int32)
            vs = load_cols(lv[j], sl)
            if j < NBUF:
                fire(lv[j + NBUF], sl)
            else:
                @pl.when(ci < CHUNKS - 1)
                def _():
                    lvn = lab_v[pl.ds(off + ci * 16 + 16, 16)]
                    fire(lvn[j - NBUF], sl)
            scale_store(vs, iv)
        return carry

    lax.fori_loop(0, CHUNKS, chunk, 0)

    base = wid * B_PER_W
    pltpu.sync_copy(out_v, out_hbm.at[:, pl.ds(base, B_PER_W)])


def kernel(labels, embedding_table):
    labels_i = labels.astype(jnp.int32)
    run = pl.kernel(
        _sc_body,
        out_type=jax.ShapeDtypeStruct((HIDDEN, BATCH), jnp.float32),
        mesh=plsc.VectorSubcoreMesh(core_axis_name="c", subcore_axis_name="s"),
        scratch_types=[
            pltpu.VMEM((1024,), jnp.int32),
            pltpu.VMEM((NBUF, HIDDEN, 128), jnp.float32),
            pltpu.VMEM((HIDDEN, B_PER_W), jnp.float32),
            pltpu.SemaphoreType.DMA((NBUF,)),
        ],
        compiler_params=pltpu.CompilerParams(
            use_tc_tiling_on_sc=True, needs_layout_passes=False
        ),
    )
    out_t = run(labels_i, embedding_table.T)
    return out_t.T
